# baseline (speedup 1.0000x reference)
import functools

import jax
import jax.numpy as jnp
from jax import lax
from jax.experimental import pallas as pl
from jax.experimental.pallas import tpu as pltpu
from jax.experimental.pallas import tpu_sc as plsc


def kernel(x, tok_emb, pos_emb):
    B, T = x.shape
    V, D = tok_emb.shape
    L = 16

    info = plsc.get_sparse_core_info()
    NC, NS = info.num_cores, info.num_subcores
    NW = NC * NS
    t_span = T // NW
    C = 16
    nch = t_span // C
    G = nch * B
    NBUF = 5
    AHEAD = NBUF - 2
    VPR = D // L

    mesh = plsc.VectorSubcoreMesh(core_axis_name="c", subcore_axis_name="s")

    @functools.partial(
        pl.kernel,
        mesh=mesh,
        out_type=jax.ShapeDtypeStruct((B * T, D), jnp.float32),
        scratch_types=[
            pltpu.VMEM((nch, B, C), jnp.int32),
            pltpu.VMEM((NBUF, C, D), jnp.float32),
            pltpu.VMEM((2, C, D), jnp.float32),
            pltpu.SemaphoreType.DMA,
            pltpu.SemaphoreType.DMA,
            pltpu.SemaphoreType.DMA,
        ],
    )
    def emb_kernel(x_hbm, tok_hbm, pos_hbm, out_hbm, idx_v, rows_v, pos_v,
                   sem_g, sem_o, sem_p):
        wid = lax.axis_index("s") * NC + lax.axis_index("c")
        t0 = wid * t_span

        pltpu.sync_copy(x_hbm.at[wid], idx_v)
        pos_cp = [
            pltpu.async_copy(pos_hbm.at[pl.ds(t0 + c * C, C)], pos_v.at[c],
                             sem_p)
            for c in range(2)
        ]

        gathers = [None] * G
        outs = [None] * G
        for g in range(AHEAD):
            ch, b = divmod(g, B)
            gathers[g] = pltpu.async_copy(
                tok_hbm.at[idx_v.at[ch, b]], rows_v.at[g % NBUF], sem_g)

        for g in range(G):
            ch, b = divmod(g, B)
            slot = g % NBUF
            gathers[g].wait()
            if b == 0:
                pos_cp[ch % 2].wait()

            @plsc.parallel_loop(0, C * VPR, unroll=8)
            def add_body(i):
                r = i // VPR
                col = (i % VPR) * L
                rows_v[slot, r, pl.ds(col, L)] = (
                    rows_v[slot, r, pl.ds(col, L)]
                    + pos_v[ch % 2, r, pl.ds(col, L)]
                )

            if b == B - 1 and ch + 2 < nch:
                pos_cp[ch % 2] = pltpu.async_copy(
                    pos_hbm.at[pl.ds(t0 + (ch + 2) * C, C)],
                    pos_v.at[ch % 2], sem_p)
            row0 = b * T + t0 + ch * C
            outs[g] = pltpu.async_copy(
                rows_v.at[slot], out_hbm.at[pl.ds(row0, C)], sem_o)

            ng = g + AHEAD
            if ng < G:
                if ng >= NBUF:
                    outs[ng - NBUF].wait()
                ch2, b2 = divmod(ng, B)
                gathers[ng] = pltpu.async_copy(
                    tok_hbm.at[idx_v.at[ch2, b2]], rows_v.at[ng % NBUF],
                    sem_g)

        for g in range(max(0, G - NBUF), G):
            outs[g].wait()

    x3 = x.reshape(B, NW, nch, C).transpose(1, 2, 0, 3)
    out = emb_kernel(x3, tok_emb, pos_emb)
    return out.reshape(B, T, D)
